# baseline (speedup 1.0000x reference)
import functools

import jax
import jax.numpy as jnp
from jax import lax
from jax.experimental import pallas as pl
from jax.experimental.pallas import tpu as pltpu
from jax.experimental.pallas import tpu_sc as plsc

_NUM_EMB = 1000000
_DIM = 64
_BATCH = 16384
_NC = 2
_NS = 16
_NW = _NC * _NS
_TAIL0 = 999936
_CPW = 31360
_CHUNK = 512
_NCH = 62
_TRASH = _BATCH
_CAP = 896

_mesh = plsc.VectorSubcoreMesh(core_axis_name="c", subcore_axis_name="s")


@functools.partial(
    pl.kernel,
    mesh=_mesh,
    out_type=jax.ShapeDtypeStruct((_BATCH + 1, 128), jnp.float32),
    scratch_types=[
        pltpu.VMEM((_BATCH + 16,), jnp.int32),
        pltpu.VMEM((_BATCH + 16,), jnp.int32),
        pltpu.VMEM((2, _DIM, _CHUNK), jnp.float32),
        pltpu.VMEM((_DIM, _DIM), jnp.float32),
        pltpu.VMEM((128, 128), jnp.float32),
        pltpu.VMEM((1, 128), jnp.int32),
        pltpu.SemaphoreType.DMA,
        pltpu.SemaphoreType.DMA,
    ],
    compiler_params=pltpu.CompilerParams(
        use_tc_tiling_on_sc=True, needs_layout_passes=False
    ),
)
def _gather_kernel(
    idx_hbm, tab_hbm, tail_hbm, out_hbm,
    hpk_v, cpk_v, slab_v, tail_v, batch_v, posb_v, sem_s, sem_o,
):
    wid = lax.axis_index("s") * _NC + lax.axis_index("c")
    lo = wid * _CPW
    hi = jnp.minimum(lo + _CPW, _NUM_EMB)
    teff = jnp.minimum(hi, _TAIL0) - lo

    iota16 = lax.iota(jnp.int32, 16)

    pltpu.sync_copy(tail_hbm, tail_v)
    pltpu.sync_copy(idx_hbm, cpk_v.at[pl.ds(0, _BATCH)])

    def scan_body(q, cnt):
        v = cpk_v[pl.ds(q * 16, 16)]
        m = (v >= lo) & (v < hi)
        pk = ((v - lo) << 14) | (iota16 + q * 16)
        plsc.store_compressed(hpk_v.at[pl.ds(cnt, 16)], pk, mask=m)
        return cnt + plsc.all_reduce_population_count(m)[0]

    cnt = lax.fori_loop(0, _BATCH // 16, scan_body, 0)
    nrescan = (cnt + 15) >> 4

    def rescan(rlo, rhi):
        def rbody(q, cc):
            h = hpk_v[pl.ds(q * 16, 16)]
            rel = h >> 14
            m2 = ((iota16 + q * 16) < cnt) & (rel >= rlo) & (rel < rhi)
            plsc.store_compressed(cpk_v.at[pl.ds(cc, 16)], h, mask=m2)
            return cc + plsc.all_reduce_population_count(m2)[0]

        return lax.fori_loop(0, nrescan, rbody, 0)

    def flush_batch():
        pltpu.async_copy(batch_v, out_hbm.at[posb_v.at[0]], sem_o).wait()

    def serve_waves(cc, app, gather_vals):
        def wbody(w, app):
            bat = app & 127
            pkv = cpk_v[pl.ds(w * 16, 16)]
            valid = (w * 16 + iota16) < cc
            posb_v[0, pl.ds(bat, 16)] = jnp.where(
                valid, pkv & (_BATCH - 1), _TRASH
            )
            col16 = jnp.where(valid, pkv >> 14, 0)
            gather_vals(col16, iota16 + bat)

            @pl.when(((app + 16) & 127) == 0)
            def _():
                flush_batch()

            return app + 16

        return lax.fori_loop(0, (cc + 15) >> 4, wbody, app)

    def slab_off(c):
        return jnp.minimum(lo + c * _CHUNK, _TAIL0 - _CHUNK)

    pltpu.async_copy(
        tab_hbm.at[:, pl.ds(slab_off(0), _CHUNK)], slab_v.at[0], sem_s
    )

    def chunk_body(c, wv):
        soff = slab_off(c)

        @pl.when(c < _NCH - 1)
        def _():
            pltpu.async_copy(
                tab_hbm.at[:, pl.ds(slab_off(c + 1), _CHUNK)],
                slab_v.at[(c + 1) & 1],
                sem_s,
            )

        pltpu.make_async_copy(
            tab_hbm.at[:, pl.ds(soff, _CHUNK)], slab_v.at[c & 1], sem_s
        ).wait()

        cc = rescan(c * _CHUNK, jnp.minimum((c + 1) * _CHUNK, teff))
        shift = lo - soff
        buf16 = jnp.full((16,), c & 1, jnp.int32)

        def gather_slab(col16, row16):
            scol = jnp.clip(col16 + shift, 0, _CHUNK - 1)
            for f in range(_DIM):
                v16 = plsc.load_gather(
                    slab_v, [buf16, jnp.full((16,), f, jnp.int32), scol]
                )
                plsc.store_scatter(
                    batch_v, [row16, jnp.full((16,), f, jnp.int32)], v16
                )

        return serve_waves(cc, wv, gather_slab)

    app = lax.fori_loop(0, _NCH, chunk_body, 0)

    cc = rescan(teff, hi - lo)

    def gather_tail(col16, row16):
        tcol = jnp.clip(col16 - teff, 0, _DIM - 1)
        for f in range(_DIM):
            v16 = plsc.load_gather(
                tail_v, [jnp.full((16,), f, jnp.int32), tcol]
            )
            plsc.store_scatter(
                batch_v, [row16, jnp.full((16,), f, jnp.int32)], v16
            )

    app = serve_waves(cc, app, gather_tail)

    @pl.when((app & 127) != 0)
    def _():
        bend = app & 127
        for q in range(8):
            vec = posb_v[0, pl.ds(q * 16, 16)]
            posb_v[0, pl.ds(q * 16, 16)] = jnp.where(
                (q * 16 + iota16) < bend, vec, _TRASH
            )
        flush_batch()


def kernel(indices, drug_feature):
    idx = indices.astype(jnp.int32)
    scr = _gather_kernel(idx, drug_feature.T, drug_feature[_TAIL0:].T)
    return scr[:_BATCH, :_DIM]


# speedup vs baseline: 1.5301x; 1.5301x over previous
import functools

import jax
import jax.numpy as jnp
from jax import lax
from jax.experimental import pallas as pl
from jax.experimental.pallas import tpu as pltpu
from jax.experimental.pallas import tpu_sc as plsc

_NUM_EMB = 1000000
_DIM = 64
_BATCH = 16384
_NC = 2
_NS = 16
_NW = _NC * _NS
_BPW = _BATCH // _NW
_CHUNK = 128
_NCHUNK = _BPW // _CHUNK

_mesh = plsc.VectorSubcoreMesh(core_axis_name="c", subcore_axis_name="s")


@functools.partial(
    pl.kernel,
    mesh=_mesh,
    out_type=jax.ShapeDtypeStruct((_BATCH, _DIM), jnp.float32),
    scratch_types=[
        pltpu.VMEM((_NCHUNK, _CHUNK), jnp.int32),
        pltpu.VMEM((_BPW, _DIM), jnp.float32),
        pltpu.SemaphoreType.DMA,
    ],
    compiler_params=pltpu.CompilerParams(use_tc_tiling_on_sc=False),
)
def _gather_kernel(idx_hbm, table_hbm, out_hbm, idx_v, rows_v, sem):
    wid = lax.axis_index("s") * _NC + lax.axis_index("c")
    pltpu.sync_copy(idx_hbm.at[pl.ds(wid * _NCHUNK, _NCHUNK)], idx_v)
    copies = [
        pltpu.async_copy(
            table_hbm.at[idx_v.at[j]],
            rows_v.at[pl.ds(j * _CHUNK, _CHUNK)],
            sem,
        )
        for j in range(_NCHUNK)
    ]
    for c in copies:
        c.wait()
    pltpu.sync_copy(rows_v, out_hbm.at[pl.ds(wid * _BPW, _BPW)])


def kernel(indices, drug_feature):
    idx = indices.astype(jnp.int32).reshape(_BATCH // _CHUNK, _CHUNK)
    return _gather_kernel(idx, drug_feature)


# speedup vs baseline: 2.4845x; 1.6238x over previous
import functools

import jax
import jax.numpy as jnp
from jax import lax
from jax.experimental import pallas as pl
from jax.experimental.pallas import tpu as pltpu
from jax.experimental.pallas import tpu_sc as plsc

_NUM_EMB = 1000000
_DIM = 64
_BATCH = 16384
_NC = 2
_NS = 16
_NW = _NC * _NS
_BPW = _BATCH // _NW
_GRP = 16
_NGRP = _BPW // _GRP

_mesh = plsc.VectorSubcoreMesh(core_axis_name="c", subcore_axis_name="s")


@functools.partial(
    pl.kernel,
    mesh=_mesh,
    out_type=jax.ShapeDtypeStruct((_DIM, _BATCH), jnp.float32),
    scratch_types=[
        pltpu.VMEM((_BPW,), jnp.int32),
        pltpu.VMEM((2, _GRP, 8, _DIM), jnp.float32),
        pltpu.VMEM((_DIM, _BPW), jnp.float32),
        pltpu.SemaphoreType.DMA,
    ],
    compiler_params=pltpu.CompilerParams(
        use_tc_tiling_on_sc=True, needs_layout_passes=False
    ),
)
def _gather_kernel(idx_hbm, tab_hbm, out_hbm, idx_v, slab_v, out_v, sem):
    wid = lax.axis_index("s") * _NC + lax.axis_index("c")
    base = wid * _BPW
    pltpu.sync_copy(idx_hbm.at[pl.ds(base, _BPW)], idx_v)

    iota16 = lax.iota(jnp.int32, 16)

    def fire(g):
        vec = idx_v[pl.ds(g * _GRP, 16)]
        for k in range(_GRP):
            tb = pl.multiple_of((vec[k] >> 3) << 3, 8)
            pltpu.async_copy(
                tab_hbm.at[pl.ds(tb, 8), :], slab_v.at[g & 1, k], sem
            )

    fire(0)

    def group_body(g, carry):
        @pl.when(g < _NGRP - 1)
        def _():
            fire(g + 1)

        for k in range(_GRP):
            pltpu.make_async_copy(
                tab_hbm.at[pl.ds(0, 8), :], slab_v.at[g & 1, k], sem
            ).wait()

        vec = idx_v[pl.ds(g * _GRP, 16)]
        sub16 = vec & 7
        buf16 = jnp.full((16,), g & 1, jnp.int32)
        for f in range(_DIM):
            v16 = plsc.load_gather(
                slab_v, [buf16, iota16, sub16, jnp.full((16,), f, jnp.int32)]
            )
            out_v[f, pl.ds(g * _GRP, 16)] = v16
        return carry

    lax.fori_loop(0, _NGRP, group_body, 0)
    pltpu.sync_copy(out_v, out_hbm.at[:, pl.ds(base, _BPW)])


def kernel(indices, drug_feature):
    idx = indices.astype(jnp.int32)
    out_t = _gather_kernel(idx, drug_feature)
    return out_t.T


# speedup vs baseline: 2.5440x; 1.0239x over previous
import functools

import jax
import jax.numpy as jnp
from jax import lax
from jax.experimental import pallas as pl
from jax.experimental.pallas import tpu as pltpu
from jax.experimental.pallas import tpu_sc as plsc

_NUM_EMB = 1000000
_DIM = 64
_BATCH = 16384
_NC = 2
_NS = 16
_NW = _NC * _NS
_BPW = _BATCH // _NW
_GRP = 16
_NGRP = _BPW // _GRP

_mesh = plsc.VectorSubcoreMesh(core_axis_name="c", subcore_axis_name="s")


@functools.partial(
    pl.kernel,
    mesh=_mesh,
    out_type=jax.ShapeDtypeStruct((_DIM, _BATCH), jnp.float32),
    scratch_types=[
        pltpu.VMEM((_BPW,), jnp.int32),
        pltpu.VMEM((4, _GRP * 8, _DIM), jnp.float32),
        pltpu.VMEM((_DIM, _BPW), jnp.float32),
        pltpu.SemaphoreType.DMA,
    ],
    compiler_params=pltpu.CompilerParams(
        use_tc_tiling_on_sc=True, needs_layout_passes=False
    ),
)
def _gather_kernel(idx_hbm, tab_hbm, out_hbm, idx_v, slab_v, out_v, sem):
    wid = lax.axis_index("s") * _NC + lax.axis_index("c")
    base = wid * _BPW
    pltpu.sync_copy(idx_hbm.at[pl.ds(base, _BPW)], idx_v)

    iota16 = lax.iota(jnp.int32, 16)
    iota8x = iota16 * 8

    def fire(g):
        vec = idx_v[pl.ds(g * _GRP, 16)]
        for k in range(_GRP):
            tb = pl.multiple_of((vec[k] >> 3) << 3, 8)
            pltpu.async_copy(
                tab_hbm.at[pl.ds(tb, 8), :],
                slab_v.at[g & 3, pl.ds(k * 8, 8)],
                sem,
            )

    fire(0)
    fire(1)
    fire(2)

    def group_body(g, carry):
        @pl.when(g < _NGRP - 3)
        def _():
            fire(g + 3)

        pltpu.make_async_copy(
            tab_hbm.at[pl.ds(0, _GRP * 8), :], slab_v.at[g & 3], sem
        ).wait()

        vec = idx_v[pl.ds(g * _GRP, 16)]
        row16 = iota8x + (vec & 7)
        buf16 = jnp.full((16,), g & 3, jnp.int32)
        for f in range(_DIM):
            v16 = plsc.load_gather(
                slab_v, [buf16, row16, jnp.full((16,), f, jnp.int32)]
            )
            out_v[f, pl.ds(g * _GRP, 16)] = v16
        return carry

    lax.fori_loop(0, _NGRP, group_body, 0)
    pltpu.sync_copy(out_v, out_hbm.at[:, pl.ds(base, _BPW)])


def kernel(indices, drug_feature):
    idx = indices.astype(jnp.int32)
    out_t = _gather_kernel(idx, drug_feature)
    return out_t.T
